# Initial kernel scaffold; baseline (speedup 1.0000x reference)
import jax
import jax.numpy as jnp
from jax.experimental import pallas as pl


def kernel(outputs, atom_split):
    raise NotImplementedError("write your pallas kernel here")


# speedup vs baseline: 4.5033x; 4.5033x over previous
import functools

import jax
import jax.numpy as jnp
from jax import lax
from jax.experimental import pallas as pl
from jax.experimental.pallas import tpu as pltpu
from jax.experimental.pallas import tpu_sc as plsc

N_ATOMS = 320000
DEPTH = 128
N_SEG = 10000
N_SEG_PAD = 10240

NUM_CORES = 2
NUM_SUBCORES = 16
NW = NUM_CORES * NUM_SUBCORES
PER_W = N_ATOMS // NW
CHUNK = 80
N_CHUNK = PER_W // CHUNK
BLOCK = 80
CPB = BLOCK // CHUNK
N_BLOCK = PER_W // BLOCK
SEG_PER_TILE = N_SEG_PAD // NUM_SUBCORES


def _sc_partial_segsum(outputs, idx3, zeros_hbm):
    mesh = plsc.VectorSubcoreMesh(core_axis_name="c", subcore_axis_name="s")

    @functools.partial(
        pl.kernel,
        out_type=jax.ShapeDtypeStruct((NUM_CORES, N_SEG_PAD, DEPTH), jnp.float32),
        mesh=mesh,
        scratch_types=[
            pltpu.VMEM((N_CHUNK, CHUNK), jnp.int32),
            pltpu.VMEM((BLOCK, DEPTH), jnp.float32),
            pltpu.VMEM_SHARED((N_SEG_PAD, DEPTH), jnp.float32),
        ],
    )
    def k(rows_hbm, idx_hbm, zero_hbm, out_hbm, idx_v, rows_v, acc_sh):
        cid = lax.axis_index("c")
        sid = lax.axis_index("s")
        wid = cid * NUM_SUBCORES + sid
        base = wid * PER_W

        pltpu.sync_copy(zero_hbm.at[pl.ds(sid * SEG_PER_TILE, SEG_PER_TILE)],
                        acc_sh.at[pl.ds(sid * SEG_PER_TILE, SEG_PER_TILE)])
        pltpu.sync_copy(idx_hbm.at[wid], idx_v)
        plsc.subcore_barrier()

        def block_body(i, carry):
            pltpu.sync_copy(rows_hbm.at[pl.ds(base + i * BLOCK, BLOCK)], rows_v)
            def chunk_body(b, carry2):
                pltpu.sync_copy(rows_v.at[pl.ds(b * CHUNK, CHUNK)],
                                acc_sh.at[idx_v.at[i * CPB + b]],
                                add=True)
                return carry2
            return lax.fori_loop(0, CPB, chunk_body, carry)

        lax.fori_loop(0, N_BLOCK, block_body, 0)
        plsc.subcore_barrier()

        pltpu.sync_copy(acc_sh.at[pl.ds(sid * SEG_PER_TILE, SEG_PER_TILE)],
                        out_hbm.at[cid].at[pl.ds(sid * SEG_PER_TILE, SEG_PER_TILE)])

    return k(outputs, idx3, zeros_hbm)


def _combine_kernel(p_ref, o_ref):
    o_ref[...] = p_ref[0] + p_ref[1]


def _combine(partials):
    grid = 10
    rows = N_SEG // grid
    return pl.pallas_call(
        _combine_kernel,
        grid=(grid,),
        in_specs=[pl.BlockSpec((NUM_CORES, rows, DEPTH), lambda i: (0, i, 0))],
        out_specs=pl.BlockSpec((rows, DEPTH), lambda i: (i, 0)),
        out_shape=jax.ShapeDtypeStruct((N_SEG, DEPTH), jnp.float32),
    )(partials)


def kernel(outputs, atom_split):
    idx3 = atom_split.astype(jnp.int32).reshape(NW, N_CHUNK, CHUNK)
    zeros_hbm = jnp.zeros((N_SEG_PAD, DEPTH), jnp.float32)
    partials = _sc_partial_segsum(outputs, idx3, zeros_hbm)
    return _combine(partials)
